# baseline (speedup 1.0000x reference)
import functools

import jax
import jax.numpy as jnp
from jax import lax
from jax.experimental import pallas as pl
from jax.experimental.pallas import tpu as pltpu
from jax.experimental.pallas import tpu_sc as plsc

VOCAB = 1000
T = 50
EMB = 32
BATCH = 1024
NC = 2
NS = 16
NW = NC * NS
HT = T // 2
SLOTS = BATCH * T // HT
ROUNDS = SLOTS // NW
SLAB_ROWS = NS * HT


def _precompute_body(tok_ref, pos_ref, w_ref, b_ref, tokl_ref, posl_ref):
    w = w_ref[...]
    tokl_ref[...] = (
        jnp.dot(tok_ref[...], w, preferred_element_type=jnp.float32) + b_ref[...]
    )
    posl_ref[...] = jnp.dot(pos_ref[...], w, preferred_element_type=jnp.float32)


def _precompute_tables(tok_table, pos_table, W, b):
    return pl.pallas_call(
        _precompute_body,
        out_shape=(
            jax.ShapeDtypeStruct((VOCAB, VOCAB), jnp.float32),
            jax.ShapeDtypeStruct((T, VOCAB), jnp.float32),
        ),
    )(tok_table, pos_table, W, b.reshape(1, VOCAB))


def _sc_gather_kernel(x_hbm, tokl_hbm, posl_hbm, sidx_hbm, out_hbm,
                      idx_v, pos_v, sidx_v, rows_v, slab, sem):
    cid = lax.axis_index("c")
    sid = lax.axis_index("s")
    w = sid * NC + cid
    pltpu.sync_copy(posl_hbm.at[cid], pos_v)
    pltpu.sync_copy(sidx_hbm.at[sid], sidx_v)
    reg = slab.at[pl.ds(sid * HT, HT)]

    def round_body(r, carry):
        g = r * NW + w
        pltpu.sync_copy(x_hbm.at[g], idx_v)
        pltpu.async_copy(tokl_hbm.at[idx_v], rows_v, sem).wait()
        pltpu.sync_copy(pos_v, reg)
        pltpu.sync_copy(rows_v, slab.at[sidx_v], add=True)
        pltpu.sync_copy(reg, out_hbm.at[pl.ds(g * HT, HT)])
        return carry

    lax.fori_loop(0, ROUNDS, round_body, 0)


_sc_gather = functools.partial(
    pl.kernel,
    mesh=plsc.VectorSubcoreMesh(core_axis_name="c", subcore_axis_name="s"),
    out_type=jax.ShapeDtypeStruct((BATCH * T, VOCAB), jnp.float32),
    scratch_types=[
        pltpu.VMEM((HT,), jnp.int32),
        pltpu.VMEM((HT, VOCAB), jnp.float32),
        pltpu.VMEM((HT,), jnp.int32),
        pltpu.VMEM((HT, VOCAB), jnp.float32),
        pltpu.VMEM_SHARED((SLAB_ROWS, VOCAB), jnp.float32),
        pltpu.SemaphoreType.DMA,
    ],
    compiler_params=pltpu.CompilerParams(use_tc_tiling_on_sc=False),
)(_sc_gather_kernel)


@jax.jit
def kernel(x, tok_table, pos_table, W, b):
    tok_logits, pos_logits = _precompute_tables(tok_table, pos_table, W, b)
    sidx = (jnp.arange(NS, dtype=jnp.int32)[:, None] * HT
            + jnp.arange(HT, dtype=jnp.int32)[None, :])
    out = _sc_gather(x.reshape(SLOTS, HT), tok_logits,
                     pos_logits.reshape(NC, HT, VOCAB), sidx)
    return out.reshape(BATCH, T, VOCAB)


# speedup vs baseline: 1.0920x; 1.0920x over previous
import functools

import jax
import jax.numpy as jnp
from jax import lax
from jax.experimental import pallas as pl
from jax.experimental.pallas import tpu as pltpu
from jax.experimental.pallas import tpu_sc as plsc

VOCAB = 1000
T = 50
EMB = 32
BATCH = 1024
NC = 2
NS = 16
NW = NC * NS
HT = T // 2
SLOTS = BATCH * T // HT
ROUNDS = SLOTS // NW
CHUNK = 63
CHUNK_LAST = VOCAB - CHUNK * (NS - 1)


def _precompute_body(tok_ref, pos_ref, w_ref, b_ref, tokl_ref, posl_ref):
    w = w_ref[...]
    tokl_ref[...] = (
        jnp.dot(tok_ref[...], w, preferred_element_type=jnp.float32) + b_ref[...]
    )
    posl_ref[...] = jnp.dot(pos_ref[...], w, preferred_element_type=jnp.float32)


def _precompute_tables(tok_table, pos_table, W, b):
    return pl.pallas_call(
        _precompute_body,
        out_shape=(
            jax.ShapeDtypeStruct((VOCAB, VOCAB), jnp.float32),
            jax.ShapeDtypeStruct((T, VOCAB), jnp.float32),
        ),
    )(tok_table, pos_table, W, b.reshape(1, VOCAB))


def _sc_gather_kernel(xt_hbm, tokl_hbm, posl_hbm, out_hbm,
                      idx_all, pos_v, rows_v, tok_slab, sem):
    cid = lax.axis_index("c")
    sid = lax.axis_index("s")
    w = sid * NC + cid
    pltpu.sync_copy(xt_hbm.at[w], idx_all)
    pltpu.sync_copy(posl_hbm.at[cid], pos_v)

    @pl.when(sid < NS - 1)
    def _load_chunk():
        pltpu.sync_copy(tokl_hbm.at[pl.ds(sid * CHUNK, CHUNK)],
                        tok_slab.at[pl.ds(sid * CHUNK, CHUNK)])

    @pl.when(sid == NS - 1)
    def _load_last():
        pltpu.sync_copy(tokl_hbm.at[pl.ds((NS - 1) * CHUNK, CHUNK_LAST)],
                        tok_slab.at[pl.ds((NS - 1) * CHUNK, CHUNK_LAST)])

    plsc.subcore_barrier()

    def round_body(r, carry):
        g = r * NW + w
        pltpu.async_copy(tok_slab.at[idx_all.at[r]], rows_v, sem).wait()

        def add_row(i, c):
            rows_v[i, :] = rows_v[i, :] + pos_v[i, :]
            return c

        lax.fori_loop(0, HT, add_row, 0)
        pltpu.sync_copy(rows_v, out_hbm.at[pl.ds(g * HT, HT)])
        return carry

    lax.fori_loop(0, ROUNDS, round_body, 0)


_sc_gather = functools.partial(
    pl.kernel,
    mesh=plsc.VectorSubcoreMesh(core_axis_name="c", subcore_axis_name="s"),
    out_type=jax.ShapeDtypeStruct((BATCH * T, VOCAB), jnp.float32),
    scratch_types=[
        pltpu.VMEM((ROUNDS, HT), jnp.int32),
        pltpu.VMEM((HT, VOCAB), jnp.float32),
        pltpu.VMEM((HT, VOCAB), jnp.float32),
        pltpu.VMEM_SHARED((VOCAB, VOCAB), jnp.float32),
        pltpu.SemaphoreType.DMA,
    ],
    compiler_params=pltpu.CompilerParams(use_tc_tiling_on_sc=False),
)(_sc_gather_kernel)


@jax.jit
def kernel(x, tok_table, pos_table, W, b):
    tok_logits, pos_logits = _precompute_tables(tok_table, pos_table, W, b)
    xt = x.reshape(SLOTS // NW, NW, HT).transpose(1, 0, 2)
    out = _sc_gather(xt, tok_logits, pos_logits.reshape(NC, HT, VOCAB))
    return out.reshape(BATCH, T, VOCAB)


# speedup vs baseline: 1.1780x; 1.0787x over previous
import functools

import jax
import jax.numpy as jnp
from jax import lax
from jax.experimental import pallas as pl
from jax.experimental.pallas import tpu as pltpu
from jax.experimental.pallas import tpu_sc as plsc

VOCAB = 1000
T = 50
EMB = 32
BATCH = 1024
NC = 2
NS = 16
HT = T // 2
SLOTS = BATCH * T // HT
ROUNDS = SLOTS // NS
CW = 512
COFF = VOCAB - CW
CHUNK = 63
CHUNK_LAST = VOCAB - CHUNK * (NS - 1)


def _precompute_body(tok_ref, pos_ref, w_ref, b_ref, tokl_ref, posl_ref):
    w = w_ref[...]
    tokl_ref[...] = (
        jnp.dot(tok_ref[...], w, preferred_element_type=jnp.float32) + b_ref[...]
    )
    posl_ref[...] = jnp.dot(pos_ref[...], w, preferred_element_type=jnp.float32)


def _precompute_tables(tok_table, pos_table, W, b):
    return pl.pallas_call(
        _precompute_body,
        out_shape=(
            jax.ShapeDtypeStruct((VOCAB, VOCAB), jnp.float32),
            jax.ShapeDtypeStruct((T, VOCAB), jnp.float32),
        ),
    )(tok_table, pos_table, W, b.reshape(1, VOCAB))


def _sc_gather_kernel(xt_hbm, tokl_hbm, posl_hbm, out_hbm,
                      idx_all, pos_v, rows_v0, rows_v1, tok_slab, gsem, wsem):
    cid = lax.axis_index("c")
    sid = lax.axis_index("s")
    c0 = cid * COFF
    pltpu.sync_copy(xt_hbm.at[sid], idx_all)
    pltpu.sync_copy(posl_hbm.at[:, pl.ds(c0, CW)], pos_v)

    @pl.when(sid < NS - 1)
    def _load_chunk():
        pltpu.sync_copy(tokl_hbm.at[pl.ds(sid * CHUNK, CHUNK), pl.ds(c0, CW)],
                        tok_slab.at[pl.ds(sid * CHUNK, CHUNK)])

    @pl.when(sid == NS - 1)
    def _load_last():
        pltpu.sync_copy(
            tokl_hbm.at[pl.ds((NS - 1) * CHUNK, CHUNK_LAST), pl.ds(c0, CW)],
            tok_slab.at[pl.ds((NS - 1) * CHUNK, CHUNK_LAST)])

    plsc.subcore_barrier()

    rows = (rows_v0, rows_v1)
    pltpu.async_copy(tok_slab.at[idx_all.at[0]], rows_v0, gsem)

    def phase(r, buf_cur, buf_nxt):
        g = r * NS + sid
        pltpu.make_async_copy(tokl_hbm.at[pl.ds(0, HT), pl.ds(c0, CW)],
                              buf_cur, gsem).wait()

        @pl.when(r >= 1)
        def _drain_write():
            pltpu.make_async_copy(
                buf_nxt, out_hbm.at[pl.ds(g * HT, HT), pl.ds(c0, CW)],
                wsem).wait()

        @pl.when(r + 1 < ROUNDS)
        def _next_gather():
            pltpu.async_copy(tok_slab.at[idx_all.at[r + 1]], buf_nxt, gsem)

        t0 = lax.rem(g, 2) * HT

        def add_row(i, c):
            buf_cur[i, :] = buf_cur[i, :] + pos_v[t0 + i, :]
            return c

        lax.fori_loop(0, HT, add_row, 0)
        pltpu.async_copy(buf_cur,
                         out_hbm.at[pl.ds(g * HT, HT), pl.ds(c0, CW)], wsem)

    def round_body(k, carry):
        phase(2 * k, rows[0], rows[1])
        phase(2 * k + 1, rows[1], rows[0])
        return carry

    lax.fori_loop(0, ROUNDS // 2, round_body, 0)
    pltpu.make_async_copy(rows[1],
                          out_hbm.at[pl.ds(0, HT), pl.ds(0, CW)], wsem).wait()


_sc_gather = functools.partial(
    pl.kernel,
    mesh=plsc.VectorSubcoreMesh(core_axis_name="c", subcore_axis_name="s"),
    out_type=jax.ShapeDtypeStruct((BATCH * T, VOCAB), jnp.float32),
    scratch_types=[
        pltpu.VMEM((ROUNDS, HT), jnp.int32),
        pltpu.VMEM((T, CW), jnp.float32),
        pltpu.VMEM((HT, CW), jnp.float32),
        pltpu.VMEM((HT, CW), jnp.float32),
        pltpu.VMEM_SHARED((VOCAB, CW), jnp.float32),
        pltpu.SemaphoreType.DMA,
        pltpu.SemaphoreType.DMA,
    ],
    compiler_params=pltpu.CompilerParams(use_tc_tiling_on_sc=False),
)(_sc_gather_kernel)


@jax.jit
def kernel(x, tok_table, pos_table, W, b):
    tok_logits, pos_logits = _precompute_tables(tok_table, pos_table, W, b)
    xt = x.reshape(ROUNDS, NS, HT).transpose(1, 0, 2)
    out = _sc_gather(xt, tok_logits, pos_logits)
    return out.reshape(BATCH, T, VOCAB)


# speedup vs baseline: 1.1859x; 1.0067x over previous
import functools

import jax
import jax.numpy as jnp
from jax import lax
from jax.experimental import pallas as pl
from jax.experimental.pallas import tpu as pltpu
from jax.experimental.pallas import tpu_sc as plsc

VOCAB = 1000
T = 50
EMB = 32
BATCH = 1024
NC = 2
NS = 16
SL = 64
SLOTS = BATCH * T // SL
ROUNDS = SLOTS // NS
CW = 512
COFF = VOCAB - CW
CHUNK = 63
CHUNK_LAST = VOCAB - CHUNK * (NS - 1)


def _precompute_body(tok_ref, pos_ref, w_ref, b_ref, tokl_ref, posl_ref):
    w = w_ref[...]
    tokl_ref[...] = (
        jnp.dot(tok_ref[...], w, preferred_element_type=jnp.float32) + b_ref[...]
    )
    posl_ref[...] = jnp.dot(pos_ref[...], w, preferred_element_type=jnp.float32)


def _precompute_tables(tok_table, pos_table, W, b):
    return pl.pallas_call(
        _precompute_body,
        out_shape=(
            jax.ShapeDtypeStruct((VOCAB, VOCAB), jnp.float32),
            jax.ShapeDtypeStruct((T, VOCAB), jnp.float32),
        ),
    )(tok_table, pos_table, W, b.reshape(1, VOCAB))


def _sc_gather_kernel(xt_hbm, tokl_hbm, posl_hbm, out_hbm,
                      idx_all, pos_v, rows_v0, rows_v1, tok_slab, gsem, wsem):
    cid = lax.axis_index("c")
    sid = lax.axis_index("s")
    c0 = cid * COFF
    pltpu.sync_copy(xt_hbm.at[sid], idx_all)
    pltpu.sync_copy(posl_hbm.at[:, pl.ds(c0, CW)], pos_v)

    @pl.when(sid < NS - 1)
    def _load_chunk():
        pltpu.sync_copy(tokl_hbm.at[pl.ds(sid * CHUNK, CHUNK), pl.ds(c0, CW)],
                        tok_slab.at[pl.ds(sid * CHUNK, CHUNK)])

    @pl.when(sid == NS - 1)
    def _load_last():
        pltpu.sync_copy(
            tokl_hbm.at[pl.ds((NS - 1) * CHUNK, CHUNK_LAST), pl.ds(c0, CW)],
            tok_slab.at[pl.ds((NS - 1) * CHUNK, CHUNK_LAST)])

    plsc.subcore_barrier()

    rows = (rows_v0, rows_v1)
    pltpu.async_copy(tok_slab.at[idx_all.at[0]], rows_v0, gsem)

    def phase(r, buf_cur, buf_nxt):
        g = r * NS + sid
        pltpu.make_async_copy(tokl_hbm.at[pl.ds(0, SL), pl.ds(c0, CW)],
                              buf_cur, gsem).wait()

        @pl.when(r >= 1)
        def _drain_write():
            pltpu.make_async_copy(
                buf_nxt, out_hbm.at[pl.ds(g * SL, SL), pl.ds(c0, CW)],
                wsem).wait()

        @pl.when(r + 1 < ROUNDS)
        def _next_gather():
            pltpu.async_copy(tok_slab.at[idx_all.at[r + 1]], buf_nxt, gsem)

        row0 = g * SL

        def add_row(i, c):
            buf_cur[i, :] = buf_cur[i, :] + pos_v[lax.rem(row0 + i, T), :]
            return c

        lax.fori_loop(0, SL, add_row, 0)
        pltpu.async_copy(buf_cur,
                         out_hbm.at[pl.ds(g * SL, SL), pl.ds(c0, CW)], wsem)

    def round_body(k, carry):
        phase(2 * k, rows[0], rows[1])
        phase(2 * k + 1, rows[1], rows[0])
        return carry

    lax.fori_loop(0, ROUNDS // 2, round_body, 0)
    pltpu.make_async_copy(rows[1],
                          out_hbm.at[pl.ds(0, SL), pl.ds(0, CW)], wsem).wait()


_sc_gather = functools.partial(
    pl.kernel,
    mesh=plsc.VectorSubcoreMesh(core_axis_name="c", subcore_axis_name="s"),
    out_type=jax.ShapeDtypeStruct((BATCH * T, VOCAB), jnp.float32),
    scratch_types=[
        pltpu.VMEM((ROUNDS, SL), jnp.int32),
        pltpu.VMEM((T, CW), jnp.float32),
        pltpu.VMEM((SL, CW), jnp.float32),
        pltpu.VMEM((SL, CW), jnp.float32),
        pltpu.VMEM_SHARED((VOCAB, CW), jnp.float32),
        pltpu.SemaphoreType.DMA,
        pltpu.SemaphoreType.DMA,
    ],
    compiler_params=pltpu.CompilerParams(use_tc_tiling_on_sc=False),
)(_sc_gather_kernel)


@jax.jit
def kernel(x, tok_table, pos_table, W, b):
    tok_logits, pos_logits = _precompute_tables(tok_table, pos_table, W, b)
    xt = x.reshape(ROUNDS, NS, SL).transpose(1, 0, 2)
    out = _sc_gather(xt, tok_logits, pos_logits)
    return out.reshape(BATCH, T, VOCAB)
